# baseline (speedup 1.0000x reference)
import functools

import jax
import jax.numpy as jnp
from jax import lax
from jax.experimental import pallas as pl
from jax.experimental.pallas import tpu as pltpu
from jax.experimental.pallas import tpu_sc as plsc

_NC = 8192
_NOBJ = 32
_DIMC = _NC + _NOBJ
_D = 64
_S = 16
_B = 32
_P = 4
_CHUNK = 2048
_NCHUNK = _NC // _CHUNK
_PD = _P * _D
_PS = _P * _S


_NW = 32
_CW = (_B * _NOBJ) // _NW
_AW = _CW * 8


def _sc_gather_body(idxc_hbm, idxa_hbm, cls2_hbm, attr2_hbm,
                    outc_hbm, outa_hbm, idx_c, idx_a, rows_c, rows_a, sem):
    wid = lax.axis_index("s") * 2 + lax.axis_index("c")
    basec = wid * _CW
    basea = wid * _AW
    pltpu.sync_copy(idxc_hbm.at[pl.ds(basec, _CW)], idx_c)
    pltpu.sync_copy(idxa_hbm.at[pl.ds(basea, _AW)], idx_a)
    cp1 = pltpu.async_copy(cls2_hbm.at[idx_c], rows_c, sem)
    cp2 = pltpu.async_copy(attr2_hbm.at[idx_a], rows_a, sem)
    cp1.wait()
    cp2.wait()
    pltpu.sync_copy(rows_c, outc_hbm.at[pl.ds(basec, _CW)])
    pltpu.sync_copy(rows_a, outa_hbm.at[pl.ds(basea, _AW)])


def _sc_gather(idxc, idxa, cls2, attr2):
    mesh = plsc.VectorSubcoreMesh(core_axis_name="c", subcore_axis_name="s")
    kern = functools.partial(
        pl.kernel,
        mesh=mesh,
        out_type=[jax.ShapeDtypeStruct((_B * _NOBJ, 2 * _D), jnp.float32),
                  jax.ShapeDtypeStruct((_B * _NOBJ * 8, 2 * _D), jnp.float32)],
        scratch_types=[
            pltpu.VMEM((_CW,), jnp.int32),
            pltpu.VMEM((_AW,), jnp.int32),
            pltpu.VMEM((_CW, 2 * _D), jnp.float32),
            pltpu.VMEM((_AW, 2 * _D), jnp.float32),
            pltpu.SemaphoreType.DMA,
        ],
    )(_sc_gather_body)
    return kern(idxc, idxa, cls2, attr2)


def _eye(n):
    return (lax.broadcasted_iota(jnp.int32, (n, n), 0)
            == lax.broadcasted_iota(jnp.int32, (n, n), 1)).astype(jnp.float32)


def _mT_of(a, s):
    return lax.dot_general(s, a, (((0,), (0,)), ((), ())),
                           preferred_element_type=jnp.float32) * (1.0 / (_D * _S))


def _sample(g, logits):
    v = g if logits is None else g + logits
    m = jnp.max(v, axis=1, keepdims=True)
    iota = lax.broadcasted_iota(jnp.int32, (_S, _NC), 1)
    idx = jnp.min(jnp.where(v == m, iota, _NC), axis=1, keepdims=True)
    return (iota == idx).astype(jnp.float32)


def _mlp(x_all, w1, b1r, w2, b2r):
    h = jax.nn.relu(jnp.dot(x_all, w1, preferred_element_type=jnp.float32)
                    + b1r)
    return jnp.dot(h, w2, preferred_element_type=jnp.float32) + b2r


def _step0_body(ctT_ref, ct_ref, g_ref, oparg_ref, w1_ref, b1_ref, w2_ref,
                b2_ref, init_ref, axon_out, attsel_out):
    del ctT_ref
    initrow = init_ref[...]
    onehots = [_sample(g_ref[:, k * _NC:(k + 1) * _NC], None)
               for k in range(_P)]
    tout_all = jnp.dot(jnp.concatenate(onehots, axis=0), ct_ref[...],
                       preferred_element_type=jnp.float32)
    gb = jnp.broadcast_to(initrow, (_PS, _D))
    x_all = jnp.concatenate([tout_all, gb, oparg_ref[0]], axis=1)
    axon_all = _mlp(x_all, w1_ref[...], b1_ref[...], w2_ref[...], b2_ref[...])
    for k in range(_P):
        axon_out[k] = axon_all[k * _S:(k + 1) * _S]
        attsel_out[k] = jnp.broadcast_to(initrow, (_S, _D))


def _step_body(ctT_ref, ct_ref, g_ref, objT_ref, axon_ref, attsel_ref,
               oparg_ref, w1_ref, b1_ref, w2_ref, b2_ref,
               axon_out, attsel_out, scal_ref):
    ctT = ctT_ref[...]
    mTs = [_mT_of(axon_ref[k], attsel_ref[k]) for k in range(_P)]
    mT_all = jnp.concatenate(mTs, axis=0)
    ones_row = jnp.ones((1, _D), jnp.float32)
    acc = jnp.zeros((_PD, _CHUNK), jnp.float32)
    for c in range(_NCHUNK):
        attT_all = jax.nn.relu(jnp.dot(mT_all,
                                       ctT[:, c * _CHUNK:(c + 1) * _CHUNK],
                                       preferred_element_type=jnp.float32))
        for k in range(_P):
            scal_ref[k:k + 1, c * _CHUNK:(c + 1) * _CHUNK] = jnp.dot(
                ones_row, attT_all[k * _D:(k + 1) * _D],
                preferred_element_type=jnp.float32)
        acc = acc + attT_all
    rowtot = jnp.sum(acc, axis=1, keepdims=True)
    eye = _eye(_D)
    onehots, grows = [], []
    for k in range(_P):
        attT_obj = jax.nn.relu(jnp.dot(mTs[k], objT_ref[k],
                                       preferred_element_type=jnp.float32))
        gcol = (rowtot[k * _D:(k + 1) * _D]
                + jnp.sum(attT_obj, axis=1, keepdims=True)) * (1.0 / _DIMC)
        grows.append(lax.dot_general(gcol, eye, (((0,), (0,)), ((), ())),
                                     preferred_element_type=jnp.float32))
        scal = scal_ref[k:k + 1, :]
        logits = jnp.log(scal / jnp.sum(scal) + 1e-12)
        onehots.append(_sample(g_ref[:, k * _NC:(k + 1) * _NC], logits))
    tout_all = jnp.dot(jnp.concatenate(onehots, axis=0), ct_ref[...],
                       preferred_element_type=jnp.float32)
    xs = []
    for k in range(_P):
        tout_k = tout_all[k * _S:(k + 1) * _S]
        attsel_out[k] = jax.nn.relu(
            lax.dot_general(tout_k, mTs[k], (((1,), (1,)), ((), ())),
                            preferred_element_type=jnp.float32))
        xs.append(jnp.concatenate(
            [tout_k, jnp.broadcast_to(grows[k], (_S, _D)),
             oparg_ref[0][k * _S:(k + 1) * _S]], axis=1))
    x_all = jnp.concatenate(xs, axis=0)
    axon_all = _mlp(x_all, w1_ref[...], b1_ref[...], w2_ref[...], b2_ref[...])
    for k in range(_P):
        axon_out[k] = axon_all[k * _S:(k + 1) * _S]


def _final_body(ctT_ref, objT_ref, axon_ref, attsel_ref, out_ref, len_ref):
    ctT = ctT_ref[...]
    mTs = [_mT_of(axon_ref[k], attsel_ref[k]) for k in range(_P)]
    mT_all = jnp.concatenate(mTs, axis=0)
    inv = jnp.ones((1, _D), jnp.float32) * (1.0 / _D)
    for c in range(_NCHUNK):
        attT_all = jax.nn.relu(jnp.dot(mT_all,
                                       ctT[:, c * _CHUNK:(c + 1) * _CHUNK],
                                       preferred_element_type=jnp.float32))
        sq = attT_all * attT_all
        for k in range(_P):
            len_ref[k:k + 1, c * _CHUNK:(c + 1) * _CHUNK] = jnp.dot(
                inv, sq[k * _D:(k + 1) * _D],
                preferred_element_type=jnp.float32)
    for k in range(_P):
        attT_obj = jax.nn.relu(jnp.dot(mTs[k], objT_ref[k],
                                       preferred_element_type=jnp.float32))
        len_ref[k:k + 1, _NC:] = jnp.dot(inv, attT_obj * attT_obj,
                                         preferred_element_type=jnp.float32)
    x = len_ref[...]
    m = jnp.max(x, axis=1, keepdims=True)
    sh = x - m
    out_ref[0] = sh - jnp.log(jnp.sum(jnp.exp(sh), axis=1, keepdims=True))


def kernel(gt_classes, gt_attributes, program, answer, class_emb_in,
           class_emb_out, attr_emb_in, attr_emb_out, concept_emb_in,
           concept_emb_out, op_emb, object_init, attention_init, W1, b1,
           W2, b2):
    del answer, class_emb_out, attr_emb_out, object_init
    f32 = jnp.float32
    B = _B
    NG = B // _P

    cls_idx = (gt_classes + 1).reshape(-1).astype(jnp.int32)
    attr_idx = (gt_attributes + 1).reshape(-1).astype(jnp.int32)
    cls2 = class_emb_in.reshape(-1, 2 * _D)
    attr2 = attr_emb_in.reshape(-1, 2 * _D)
    pc, pa = _sc_gather(cls_idx >> 1, attr_idx >> 1, cls2, attr2)
    parc = (cls_idx & 1)[:, None].astype(f32)
    para = (attr_idx & 1)[:, None].astype(f32)
    crow = pc[:, :_D] * (1 - parc) + pc[:, _D:] * parc
    arow = pa[:, :_D] * (1 - para) + pa[:, _D:] * para
    obj_in = (crow + arow.reshape(-1, 8, _D).sum(1)).reshape(B, _NOBJ, _D)
    objT = jnp.transpose(obj_in, (0, 2, 1))
    ctT = concept_emb_in.T
    operations = jnp.take(op_emb, program[:, :, 0], axis=0)
    arguments = jnp.take(concept_emb_out, program[:, :, 1], axis=0)
    opargs = jnp.concatenate([operations, arguments], axis=2)
    opargs = jnp.broadcast_to(
        opargs.transpose(1, 0, 2)[:, :, None, :], (4, B, _S, 96)
    ).reshape(4, NG, _PS, 96)
    b1r, b2r = b1[None], b2[None]
    initrow = attention_init[None]

    skey = jax.random.key(42)
    gs = [jax.random.gumbel(jax.random.fold_in(skey, i), (_S, B * _NC), f32)
          for i in range(4)]

    const_spec = pl.BlockSpec((_D, _NC), lambda i: (0, 0))
    ct_spec = pl.BlockSpec((_NC, _D), lambda i: (0, 0))
    g_spec = pl.BlockSpec((_S, _P * _NC), lambda i: (0, i))
    slab3 = lambda shp: pl.BlockSpec(shp, lambda i: (i, 0, 0))
    full = lambda shp: pl.BlockSpec(shp, lambda i: (0,) * len(shp))
    state_shape = jax.ShapeDtypeStruct((B, _S, _D), f32)
    state_spec = slab3((_P, _S, _D))
    oparg_spec = slab3((1, _PS, 96))

    step0 = pl.pallas_call(
        _step0_body,
        grid=(NG,),
        in_specs=[const_spec, ct_spec, g_spec, oparg_spec,
                  full((224, 256)), full((1, 256)), full((256, 64)),
                  full((1, 64)), full((1, _D))],
        out_specs=[state_spec, state_spec],
        out_shape=[state_shape, state_shape],
    )
    axon, attsel = step0(ctT, concept_emb_in, gs[0], opargs[0], W1, b1r,
                         W2, b2r, initrow)

    step = pl.pallas_call(
        _step_body,
        grid=(NG,),
        in_specs=[const_spec, ct_spec, g_spec, slab3((_P, _D, _NOBJ)),
                  state_spec, state_spec, oparg_spec,
                  full((224, 256)), full((1, 256)), full((256, 64)),
                  full((1, 64))],
        out_specs=[state_spec, state_spec],
        out_shape=[state_shape, state_shape],
        scratch_shapes=[pltpu.VMEM((_P, _NC), f32)],
    )
    for i in range(1, 4):
        axon, attsel = step(ctT, concept_emb_in, gs[i], objT, axon, attsel,
                            opargs[i], W1, b1r, W2, b2r)

    final = pl.pallas_call(
        _final_body,
        grid=(NG,),
        in_specs=[const_spec, slab3((_P, _D, _NOBJ)), state_spec, state_spec],
        out_specs=pl.BlockSpec((1, _P, _DIMC), lambda i: (i, 0, 0)),
        out_shape=jax.ShapeDtypeStruct((NG, _P, _DIMC), f32),
        scratch_shapes=[pltpu.VMEM((_P, _DIMC), f32)],
    )
    return final(ctT, objT, axon, attsel).reshape(B, _DIMC)


# speedup vs baseline: 1.0014x; 1.0014x over previous
import functools

import jax
import jax.numpy as jnp
from jax import lax
from jax.experimental import pallas as pl
from jax.experimental.pallas import tpu as pltpu
from jax.experimental.pallas import tpu_sc as plsc

_NC = 8192
_NOBJ = 32
_DIMC = _NC + _NOBJ
_D = 64
_S = 16
_B = 32
_P = 4
_CHUNK = 2048
_NCHUNK = _NC // _CHUNK
_PD = _P * _D
_PS = _P * _S


_NW = 32
_CW = (_B * _NOBJ) // _NW
_AW = _CW * 8


def _sc_gather_body(idxc_hbm, idxa_hbm, cls2_hbm, attr2_hbm,
                    outc_hbm, outa_hbm, idx_c, idx_a, rows_c, rows_a, sem):
    wid = lax.axis_index("s") * 2 + lax.axis_index("c")
    basec = wid * _CW
    basea = wid * _AW
    pltpu.sync_copy(idxc_hbm.at[pl.ds(basec, _CW)], idx_c)
    pltpu.sync_copy(idxa_hbm.at[pl.ds(basea, _AW)], idx_a)
    cp1 = pltpu.async_copy(cls2_hbm.at[idx_c], rows_c, sem)
    cp2 = pltpu.async_copy(attr2_hbm.at[idx_a], rows_a, sem)
    cp1.wait()
    cp2.wait()
    pltpu.sync_copy(rows_c, outc_hbm.at[pl.ds(basec, _CW)])
    pltpu.sync_copy(rows_a, outa_hbm.at[pl.ds(basea, _AW)])


def _sc_gather(idxc, idxa, cls2, attr2):
    mesh = plsc.VectorSubcoreMesh(core_axis_name="c", subcore_axis_name="s")
    kern = functools.partial(
        pl.kernel,
        mesh=mesh,
        out_type=[jax.ShapeDtypeStruct((_B * _NOBJ, 2 * _D), jnp.float32),
                  jax.ShapeDtypeStruct((_B * _NOBJ * 8, 2 * _D), jnp.float32)],
        scratch_types=[
            pltpu.VMEM((_CW,), jnp.int32),
            pltpu.VMEM((_AW,), jnp.int32),
            pltpu.VMEM((_CW, 2 * _D), jnp.float32),
            pltpu.VMEM((_AW, 2 * _D), jnp.float32),
            pltpu.SemaphoreType.DMA,
        ],
    )(_sc_gather_body)
    return kern(idxc, idxa, cls2, attr2)


def _eye(n):
    return (lax.broadcasted_iota(jnp.int32, (n, n), 0)
            == lax.broadcasted_iota(jnp.int32, (n, n), 1)).astype(jnp.float32)


def _mT_of(a, s):
    return lax.dot_general(s, a, (((0,), (0,)), ((), ())),
                           preferred_element_type=jnp.float32) * (1.0 / (_D * _S))


def _sample(g, logits):
    v = g if logits is None else g + logits
    m = jnp.max(v, axis=1, keepdims=True)
    iota = lax.broadcasted_iota(jnp.int32, (_S, _NC), 1)
    idx = jnp.min(jnp.where(v == m, iota, _NC), axis=1, keepdims=True)
    return (iota == idx).astype(jnp.float32)


def _mlp(x_all, w1, b1r, w2, b2r):
    h = jax.nn.relu(jnp.dot(x_all, w1, preferred_element_type=jnp.float32)
                    + b1r)
    return jnp.dot(h, w2, preferred_element_type=jnp.float32) + b2r


def _step0_body(ct_ref, g_ref, oparg_ref, w1_ref, b1_ref, w2_ref,
                b2_ref, init_ref, axon_out, attsel_out):
    initrow = init_ref[...]
    onehots = [_sample(g_ref[:, k * _NC:(k + 1) * _NC], None)
               for k in range(_P)]
    tout_all = jnp.dot(jnp.concatenate(onehots, axis=0), ct_ref[...],
                       preferred_element_type=jnp.float32)
    gb = jnp.broadcast_to(initrow, (_PS, _D))
    x_all = jnp.concatenate([tout_all, gb, oparg_ref[0]], axis=1)
    axon_all = _mlp(x_all, w1_ref[...], b1_ref[...], w2_ref[...], b2_ref[...])
    for k in range(_P):
        axon_out[k] = axon_all[k * _S:(k + 1) * _S]
        attsel_out[k] = jnp.broadcast_to(initrow, (_S, _D))


def _step_body(ctT_ref, ct_ref, g_ref, objT_ref, axon_ref, attsel_ref,
               oparg_ref, w1_ref, b1_ref, w2_ref, b2_ref,
               axon_out, attsel_out, scal_ref):
    ctT = ctT_ref[...]
    mTs = [_mT_of(axon_ref[k], attsel_ref[k]) for k in range(_P)]
    mT_all = jnp.concatenate(mTs, axis=0)
    ones_row = jnp.ones((1, _D), jnp.float32)
    acc = jnp.zeros((_PD, _CHUNK), jnp.float32)
    for c in range(_NCHUNK):
        attT_all = jax.nn.relu(jnp.dot(mT_all,
                                       ctT[:, c * _CHUNK:(c + 1) * _CHUNK],
                                       preferred_element_type=jnp.float32))
        for k in range(_P):
            scal_ref[k:k + 1, c * _CHUNK:(c + 1) * _CHUNK] = jnp.dot(
                ones_row, attT_all[k * _D:(k + 1) * _D],
                preferred_element_type=jnp.float32)
        acc = acc + attT_all
    rowtot = jnp.sum(acc, axis=1, keepdims=True)
    eye = _eye(_D)
    onehots, grows = [], []
    for k in range(_P):
        attT_obj = jax.nn.relu(jnp.dot(mTs[k], objT_ref[k],
                                       preferred_element_type=jnp.float32))
        gcol = (rowtot[k * _D:(k + 1) * _D]
                + jnp.sum(attT_obj, axis=1, keepdims=True)) * (1.0 / _DIMC)
        grows.append(lax.dot_general(gcol, eye, (((0,), (0,)), ((), ())),
                                     preferred_element_type=jnp.float32))
        scal = scal_ref[k:k + 1, :]
        logits = jnp.log(scal / jnp.sum(scal) + 1e-12)
        onehots.append(_sample(g_ref[:, k * _NC:(k + 1) * _NC], logits))
    tout_all = jnp.dot(jnp.concatenate(onehots, axis=0), ct_ref[...],
                       preferred_element_type=jnp.float32)
    xs = []
    for k in range(_P):
        tout_k = tout_all[k * _S:(k + 1) * _S]
        attsel_out[k] = jax.nn.relu(
            lax.dot_general(tout_k, mTs[k], (((1,), (1,)), ((), ())),
                            preferred_element_type=jnp.float32))
        xs.append(jnp.concatenate(
            [tout_k, jnp.broadcast_to(grows[k], (_S, _D)),
             oparg_ref[0][k * _S:(k + 1) * _S]], axis=1))
    x_all = jnp.concatenate(xs, axis=0)
    axon_all = _mlp(x_all, w1_ref[...], b1_ref[...], w2_ref[...], b2_ref[...])
    for k in range(_P):
        axon_out[k] = axon_all[k * _S:(k + 1) * _S]


def _final_body(ctT_ref, objT_ref, axon_ref, attsel_ref, out_ref, len_ref):
    ctT = ctT_ref[...]
    mTs = [_mT_of(axon_ref[k], attsel_ref[k]) for k in range(_P)]
    mT_all = jnp.concatenate(mTs, axis=0)
    inv = jnp.ones((1, _D), jnp.float32) * (1.0 / _D)
    for c in range(_NCHUNK):
        attT_all = jax.nn.relu(jnp.dot(mT_all,
                                       ctT[:, c * _CHUNK:(c + 1) * _CHUNK],
                                       preferred_element_type=jnp.float32))
        sq = attT_all * attT_all
        for k in range(_P):
            len_ref[k:k + 1, c * _CHUNK:(c + 1) * _CHUNK] = jnp.dot(
                inv, sq[k * _D:(k + 1) * _D],
                preferred_element_type=jnp.float32)
    for k in range(_P):
        attT_obj = jax.nn.relu(jnp.dot(mTs[k], objT_ref[k],
                                       preferred_element_type=jnp.float32))
        len_ref[k:k + 1, _NC:] = jnp.dot(inv, attT_obj * attT_obj,
                                         preferred_element_type=jnp.float32)
    x = len_ref[...]
    m = jnp.max(x, axis=1, keepdims=True)
    sh = x - m
    out_ref[0] = sh - jnp.log(jnp.sum(jnp.exp(sh), axis=1, keepdims=True))


def kernel(gt_classes, gt_attributes, program, answer, class_emb_in,
           class_emb_out, attr_emb_in, attr_emb_out, concept_emb_in,
           concept_emb_out, op_emb, object_init, attention_init, W1, b1,
           W2, b2):
    del answer, class_emb_out, attr_emb_out, object_init
    f32 = jnp.float32
    B = _B
    NG = B // _P

    cls_idx = (gt_classes + 1).reshape(-1).astype(jnp.int32)
    attr_idx = (gt_attributes + 1).reshape(-1).astype(jnp.int32)
    cls2 = class_emb_in.reshape(-1, 2 * _D)
    attr2 = attr_emb_in.reshape(-1, 2 * _D)
    pc, pa = _sc_gather(cls_idx >> 1, attr_idx >> 1, cls2, attr2)
    parc = (cls_idx & 1)[:, None].astype(f32)
    para = (attr_idx & 1)[:, None].astype(f32)
    crow = pc[:, :_D] * (1 - parc) + pc[:, _D:] * parc
    arow = pa[:, :_D] * (1 - para) + pa[:, _D:] * para
    obj_in = (crow + arow.reshape(-1, 8, _D).sum(1)).reshape(B, _NOBJ, _D)
    objT = jnp.transpose(obj_in, (0, 2, 1))
    ctT = concept_emb_in.T
    operations = jnp.take(op_emb, program[:, :, 0], axis=0)
    arguments = jnp.take(concept_emb_out, program[:, :, 1], axis=0)
    opargs = jnp.concatenate([operations, arguments], axis=2)
    opargs = jnp.broadcast_to(
        opargs.transpose(1, 0, 2)[:, :, None, :], (4, B, _S, 96)
    ).reshape(4, NG, _PS, 96)
    b1r, b2r = b1[None], b2[None]
    initrow = attention_init[None]

    skey = jax.random.key(42)
    gs = [jax.random.gumbel(jax.random.fold_in(skey, i), (_S, B * _NC), f32)
          for i in range(4)]

    const_spec = pl.BlockSpec((_D, _NC), lambda i: (0, 0))
    ct_spec = pl.BlockSpec((_NC, _D), lambda i: (0, 0))
    g_spec = pl.BlockSpec((_S, _P * _NC), lambda i: (0, i))
    slab3 = lambda shp: pl.BlockSpec(shp, lambda i: (i, 0, 0))
    full = lambda shp: pl.BlockSpec(shp, lambda i: (0,) * len(shp))
    state_shape = jax.ShapeDtypeStruct((B, _S, _D), f32)
    state_spec = slab3((_P, _S, _D))
    oparg_spec = slab3((1, _PS, 96))

    step0 = pl.pallas_call(
        _step0_body,
        grid=(NG,),
        in_specs=[ct_spec, g_spec, oparg_spec,
                  full((224, 256)), full((1, 256)), full((256, 64)),
                  full((1, 64)), full((1, _D))],
        out_specs=[state_spec, state_spec],
        out_shape=[state_shape, state_shape],
    )
    axon, attsel = step0(concept_emb_in, gs[0], opargs[0], W1, b1r,
                         W2, b2r, initrow)

    step = pl.pallas_call(
        _step_body,
        grid=(NG,),
        in_specs=[const_spec, ct_spec, g_spec, slab3((_P, _D, _NOBJ)),
                  state_spec, state_spec, oparg_spec,
                  full((224, 256)), full((1, 256)), full((256, 64)),
                  full((1, 64))],
        out_specs=[state_spec, state_spec],
        out_shape=[state_shape, state_shape],
        scratch_shapes=[pltpu.VMEM((_P, _NC), f32)],
    )
    for i in range(1, 4):
        axon, attsel = step(ctT, concept_emb_in, gs[i], objT, axon, attsel,
                            opargs[i], W1, b1r, W2, b2r)

    final = pl.pallas_call(
        _final_body,
        grid=(NG,),
        in_specs=[const_spec, slab3((_P, _D, _NOBJ)), state_spec, state_spec],
        out_specs=pl.BlockSpec((1, _P, _DIMC), lambda i: (i, 0, 0)),
        out_shape=jax.ShapeDtypeStruct((NG, _P, _DIMC), f32),
        scratch_shapes=[pltpu.VMEM((_P, _DIMC), f32)],
    )
    return final(ctT, objT, axon, attsel).reshape(B, _DIMC)
